# Initial kernel scaffold; baseline (speedup 1.0000x reference)
import jax
import jax.numpy as jnp
from jax.experimental import pallas as pl


def kernel(x, edge_idx, We1, be1, We2, be2, Wn1, bn1, Wn2, bn2, Wn3, bn3):
    raise NotImplementedError("write your pallas kernel here")


# speedup vs baseline: 2.9424x; 2.9424x over previous
import functools

import jax
import jax.numpy as jnp
from jax import lax
from jax.experimental import pallas as pl
from jax.experimental.pallas import tpu as pltpu
from jax.experimental.pallas import tpu_sc as plsc

N = 10000
E = 320000
D = 128

NC = 2
NS = 16
NW = NC * NS
EPW = E // NW
GC = 80
NCHUNK = EPW // GC
STRIPE = 632
STRIPE_LAST = N - (NS - 1) * STRIPE

_sc_mesh = plsc.VectorSubcoreMesh(core_axis_name="c", subcore_axis_name="s")


@functools.partial(
    pl.kernel,
    out_type=(
        jax.ShapeDtypeStruct((E, D), jnp.float32),
        jax.ShapeDtypeStruct((E, D), jnp.float32),
    ),
    mesh=_sc_mesh,
    scratch_types=[
        pltpu.VMEM((GC,), jnp.int32),
        pltpu.VMEM((GC, D), jnp.float32),
        pltpu.VMEM((GC,), jnp.int32),
        pltpu.VMEM((GC, D), jnp.float32),
        pltpu.SemaphoreType.DMA,
        pltpu.SemaphoreType.DMA,
    ],
)
def _sc_gather(xf_hbm, row_hbm, col_hbm, a_hbm, b_hbm,
               ridx_v, arows_v, cidx_v, brows_v, sem_a, sem_b):
    c = lax.axis_index("c")
    s = lax.axis_index("s")
    wid = s * NC + c
    base = wid * EPW

    def step(i, carry):
        off = base + i * GC
        pltpu.sync_copy(row_hbm.at[pl.ds(off, GC)], ridx_v)
        pltpu.sync_copy(col_hbm.at[pl.ds(off, GC)], cidx_v)
        ca = pltpu.async_copy(xf_hbm.at[ridx_v], arows_v, sem_a)
        cb = pltpu.async_copy(xf_hbm.at[cidx_v], brows_v, sem_b)
        ca.wait()
        cb.wait()
        pltpu.sync_copy(arows_v, a_hbm.at[pl.ds(off, GC)])
        pltpu.sync_copy(brows_v, b_hbm.at[pl.ds(off, GC)])
        return carry

    lax.fori_loop(0, NCHUNK, step, 0)


@functools.partial(
    pl.kernel,
    out_type=jax.ShapeDtypeStruct((2 * N, D), jnp.float32),
    mesh=_sc_mesh,
    scratch_types=[
        pltpu.VMEM((GC,), jnp.int32),
        pltpu.VMEM((GC, D), jnp.float32),
        pltpu.VMEM_SHARED((N, D), jnp.float32),
    ],
)
def _sc_scatter(m_hbm, row_hbm, zeros_hbm, out_hbm, idx_v, m_v, agg_sh):
    c = lax.axis_index("c")
    s = lax.axis_index("s")
    wid = s * NC + c
    base = wid * EPW

    @pl.when(s < NS - 1)
    def _():
        pltpu.sync_copy(zeros_hbm.at[pl.ds(s * STRIPE, STRIPE)],
                        agg_sh.at[pl.ds(s * STRIPE, STRIPE)])

    @pl.when(s == NS - 1)
    def _():
        pltpu.sync_copy(zeros_hbm.at[pl.ds(s * STRIPE, STRIPE_LAST)],
                        agg_sh.at[pl.ds(s * STRIPE, STRIPE_LAST)])

    plsc.subcore_barrier()

    def step(i, carry):
        off = base + i * GC
        pltpu.sync_copy(row_hbm.at[pl.ds(off, GC)], idx_v)
        pltpu.sync_copy(m_hbm.at[pl.ds(off, GC)], m_v)
        pltpu.sync_copy(m_v, agg_sh.at[idx_v], add=True)
        return carry

    lax.fori_loop(0, NCHUNK, step, 0)
    plsc.subcore_barrier()

    @pl.when(s < NS - 1)
    def _():
        pltpu.sync_copy(agg_sh.at[pl.ds(s * STRIPE, STRIPE)],
                        out_hbm.at[pl.ds(c * N + s * STRIPE, STRIPE)])

    @pl.when(s == NS - 1)
    def _():
        pltpu.sync_copy(agg_sh.at[pl.ds(s * STRIPE, STRIPE_LAST)],
                        out_hbm.at[pl.ds(c * N + s * STRIPE, STRIPE_LAST)])


def _softplus(z):
    return jnp.maximum(z, 0.0) + jnp.log1p(jnp.exp(-jnp.abs(z)))


BE = 2000


def _edge_mlp_body(a_ref, b_ref, w1a_ref, w1b_ref, b1_ref, w2_ref, b2_ref,
                   o_ref):
    z = jnp.dot(a_ref[...], w1a_ref[...], preferred_element_type=jnp.float32)
    z = z + jnp.dot(b_ref[...], w1b_ref[...],
                    preferred_element_type=jnp.float32)
    h = _softplus(z + b1_ref[...])
    h = jnp.dot(h, w2_ref[...], preferred_element_type=jnp.float32)
    o_ref[...] = _softplus(h + b2_ref[...])


def _edge_mlp(a, b, w1a, w1b, b1, w2, b2):
    grid = (E // BE,)
    blk = lambda i: (i, 0)
    full = lambda i: (0, 0)
    return pl.pallas_call(
        _edge_mlp_body,
        grid=grid,
        in_specs=[
            pl.BlockSpec((BE, D), blk),
            pl.BlockSpec((BE, D), blk),
            pl.BlockSpec((D, D), full),
            pl.BlockSpec((D, D), full),
            pl.BlockSpec((1, D), full),
            pl.BlockSpec((D, D), full),
            pl.BlockSpec((1, D), full),
        ],
        out_specs=pl.BlockSpec((BE, D), blk),
        out_shape=jax.ShapeDtypeStruct((E, D), jnp.float32),
    )(a, b, w1a, w1b, b1, w2, b2)


BN = 1000


def _node_mlp_body(x_ref, g0_ref, g1_ref, w1x_ref, w1g_ref, b1_ref,
                   w2_ref, b2_ref, w3_ref, b3_ref, o_ref):
    g = g0_ref[...] + g1_ref[...]
    z = jnp.dot(x_ref[...], w1x_ref[...], preferred_element_type=jnp.float32)
    z = z + jnp.dot(g, w1g_ref[...], preferred_element_type=jnp.float32)
    h = _softplus(z + b1_ref[...])
    h = jnp.dot(h, w2_ref[...], preferred_element_type=jnp.float32)
    h = _softplus(h + b2_ref[...])
    h = jnp.dot(h, w3_ref[...], preferred_element_type=jnp.float32)
    o_ref[...] = h + b3_ref[...]


def _node_mlp(xf, g0, g1, w1x, w1g, b1, w2, b2, w3, b3):
    grid = (N // BN,)
    blk = lambda i: (i, 0)
    full = lambda i: (0, 0)
    return pl.pallas_call(
        _node_mlp_body,
        grid=grid,
        in_specs=[
            pl.BlockSpec((BN, D), blk),
            pl.BlockSpec((BN, D), blk),
            pl.BlockSpec((BN, D), blk),
            pl.BlockSpec((D, D), full),
            pl.BlockSpec((D, D), full),
            pl.BlockSpec((1, D), full),
            pl.BlockSpec((D, D), full),
            pl.BlockSpec((1, D), full),
            pl.BlockSpec((D, D), full),
            pl.BlockSpec((1, D), full),
        ],
        out_specs=pl.BlockSpec((BN, D), blk),
        out_shape=jax.ShapeDtypeStruct((N, D), jnp.float32),
    )(xf, g0, g1, w1x, w1g, b1, w2, b2, w3, b3)


def kernel(x, edge_idx, We1, be1, We2, be2, Wn1, bn1, Wn2, bn2, Wn3, bn3):
    xf = x.reshape(N, D)
    row = edge_idx[0].astype(jnp.int32)
    col = edge_idx[1].astype(jnp.int32)

    a, b = _sc_gather(xf, row, col)
    m = _edge_mlp(a, b, We1[:D], We1[D:], be1.reshape(1, D),
                  We2, be2.reshape(1, D))
    parts = _sc_scatter(m, row, jnp.zeros((N, D), jnp.float32))
    out = _node_mlp(xf, parts[:N], parts[N:], Wn1[:D], Wn1[D:],
                    bn1.reshape(1, D), Wn2, bn2.reshape(1, D),
                    Wn3, bn3.reshape(1, D))
    return out.reshape(1, N, D)


# speedup vs baseline: 3.9687x; 1.3488x over previous
import functools

import jax
import jax.numpy as jnp
from jax import lax
from jax.experimental import pallas as pl
from jax.experimental.pallas import tpu as pltpu
from jax.experimental.pallas import tpu_sc as plsc

N = 10000
E = 320000
D = 128

NC = 2
NS = 16
NW = NC * NS
EPW = E // NW
GC = 80
NCHUNK = EPW // GC
NPAIR = (NCHUNK - 1) // 2
STRIPE = 632
STRIPE_LAST = N - (NS - 1) * STRIPE

_sc_mesh = plsc.VectorSubcoreMesh(core_axis_name="c", subcore_axis_name="s")


@functools.partial(
    pl.kernel,
    out_type=(
        jax.ShapeDtypeStruct((E, D), jnp.float32),
        jax.ShapeDtypeStruct((E, D), jnp.float32),
    ),
    mesh=_sc_mesh,
    scratch_types=[
        pltpu.VMEM((GC,), jnp.int32),
        pltpu.VMEM((GC,), jnp.int32),
        pltpu.VMEM((GC, D), jnp.float32),
        pltpu.VMEM((GC, D), jnp.float32),
        pltpu.VMEM((GC,), jnp.int32),
        pltpu.VMEM((GC,), jnp.int32),
        pltpu.VMEM((GC, D), jnp.float32),
        pltpu.VMEM((GC, D), jnp.float32),
        pltpu.SemaphoreType.DMA,
        pltpu.SemaphoreType.DMA,
        pltpu.SemaphoreType.DMA,
        pltpu.SemaphoreType.DMA,
    ],
)
def _sc_gather(xf_hbm, row_hbm, col_hbm, a_hbm, b_hbm,
               ridx0, cidx0, arows0, brows0,
               ridx1, cidx1, arows1, brows1,
               sem_a0, sem_b0, sem_a1, sem_b1):
    c = lax.axis_index("c")
    s = lax.axis_index("s")
    wid = s * NC + c
    base = wid * EPW

    def fire(chunk, ridx, cidx, arows, brows, sem_a, sem_b):
        off = base + chunk * GC
        pltpu.sync_copy(row_hbm.at[pl.ds(off, GC)], ridx)
        pltpu.sync_copy(col_hbm.at[pl.ds(off, GC)], cidx)
        pltpu.async_copy(xf_hbm.at[ridx], arows, sem_a)
        pltpu.async_copy(xf_hbm.at[cidx], brows, sem_b)

    def drain(chunk, ridx, cidx, arows, brows, sem_a, sem_b):
        off = base + chunk * GC
        pltpu.make_async_copy(xf_hbm.at[ridx], arows, sem_a).wait()
        pltpu.make_async_copy(xf_hbm.at[cidx], brows, sem_b).wait()
        pltpu.sync_copy(arows, a_hbm.at[pl.ds(off, GC)])
        pltpu.sync_copy(brows, b_hbm.at[pl.ds(off, GC)])

    fire(0, ridx0, cidx0, arows0, brows0, sem_a0, sem_b0)

    def step(j, carry):
        fire(2 * j + 1, ridx1, cidx1, arows1, brows1, sem_a1, sem_b1)
        drain(2 * j, ridx0, cidx0, arows0, brows0, sem_a0, sem_b0)
        fire(2 * j + 2, ridx0, cidx0, arows0, brows0, sem_a0, sem_b0)
        drain(2 * j + 1, ridx1, cidx1, arows1, brows1, sem_a1, sem_b1)
        return carry

    lax.fori_loop(0, NPAIR, step, 0)
    drain(NCHUNK - 1, ridx0, cidx0, arows0, brows0, sem_a0, sem_b0)


@functools.partial(
    pl.kernel,
    out_type=jax.ShapeDtypeStruct((2 * N, D), jnp.float32),
    mesh=_sc_mesh,
    scratch_types=[
        pltpu.VMEM((GC,), jnp.int32),
        pltpu.VMEM((GC, D), jnp.float32),
        pltpu.VMEM((GC,), jnp.int32),
        pltpu.VMEM((GC, D), jnp.float32),
        pltpu.VMEM_SHARED((N, D), jnp.float32),
        pltpu.SemaphoreType.DMA,
        pltpu.SemaphoreType.DMA,
        pltpu.SemaphoreType.DMA,
        pltpu.SemaphoreType.DMA,
    ],
)
def _sc_scatter(m_hbm, row_hbm, zeros_hbm, out_hbm,
                idx0, m0, idx1, m1, agg_sh,
                sem_i0, sem_m0, sem_i1, sem_m1):
    c = lax.axis_index("c")
    s = lax.axis_index("s")
    wid = s * NC + c
    base = wid * EPW

    @pl.when(s < NS - 1)
    def _():
        pltpu.sync_copy(zeros_hbm.at[pl.ds(s * STRIPE, STRIPE)],
                        agg_sh.at[pl.ds(s * STRIPE, STRIPE)])

    @pl.when(s == NS - 1)
    def _():
        pltpu.sync_copy(zeros_hbm.at[pl.ds(s * STRIPE, STRIPE_LAST)],
                        agg_sh.at[pl.ds(s * STRIPE, STRIPE_LAST)])

    plsc.subcore_barrier()

    def fire(chunk, idx, m, sem_i, sem_m):
        off = base + chunk * GC
        pltpu.async_copy(row_hbm.at[pl.ds(off, GC)], idx, sem_i)
        pltpu.async_copy(m_hbm.at[pl.ds(off, GC)], m, sem_m)

    def drain(chunk, idx, m, sem_i, sem_m):
        off = base + chunk * GC
        pltpu.make_async_copy(row_hbm.at[pl.ds(off, GC)], idx, sem_i).wait()
        pltpu.make_async_copy(m_hbm.at[pl.ds(off, GC)], m, sem_m).wait()
        pltpu.sync_copy(m, agg_sh.at[idx], add=True)

    fire(0, idx0, m0, sem_i0, sem_m0)

    def step(j, carry):
        fire(2 * j + 1, idx1, m1, sem_i1, sem_m1)
        drain(2 * j, idx0, m0, sem_i0, sem_m0)
        fire(2 * j + 2, idx0, m0, sem_i0, sem_m0)
        drain(2 * j + 1, idx1, m1, sem_i1, sem_m1)
        return carry

    lax.fori_loop(0, NPAIR, step, 0)
    drain(NCHUNK - 1, idx0, m0, sem_i0, sem_m0)

    plsc.subcore_barrier()

    @pl.when(s < NS - 1)
    def _():
        pltpu.sync_copy(agg_sh.at[pl.ds(s * STRIPE, STRIPE)],
                        out_hbm.at[pl.ds(c * N + s * STRIPE, STRIPE)])

    @pl.when(s == NS - 1)
    def _():
        pltpu.sync_copy(agg_sh.at[pl.ds(s * STRIPE, STRIPE_LAST)],
                        out_hbm.at[pl.ds(c * N + s * STRIPE, STRIPE_LAST)])


def _softplus(z):
    return jnp.maximum(z, 0.0) + jnp.log1p(jnp.exp(-jnp.abs(z)))


BE = 2000


def _edge_mlp_body(a_ref, b_ref, w1a_ref, w1b_ref, b1_ref, w2_ref, b2_ref,
                   o_ref):
    a = a_ref[...].astype(jnp.bfloat16)
    b = b_ref[...].astype(jnp.bfloat16)
    z = jnp.dot(a, w1a_ref[...], preferred_element_type=jnp.float32)
    z = z + jnp.dot(b, w1b_ref[...], preferred_element_type=jnp.float32)
    h = _softplus(z + b1_ref[...])
    h = jnp.dot(h.astype(jnp.bfloat16), w2_ref[...],
                preferred_element_type=jnp.float32)
    o_ref[...] = _softplus(h + b2_ref[...])


def _edge_mlp(a, b, w1a, w1b, b1, w2, b2):
    grid = (E // BE,)
    blk = lambda i: (i, 0)
    full = lambda i: (0, 0)
    return pl.pallas_call(
        _edge_mlp_body,
        grid=grid,
        in_specs=[
            pl.BlockSpec((BE, D), blk),
            pl.BlockSpec((BE, D), blk),
            pl.BlockSpec((D, D), full),
            pl.BlockSpec((D, D), full),
            pl.BlockSpec((1, D), full),
            pl.BlockSpec((D, D), full),
            pl.BlockSpec((1, D), full),
        ],
        out_specs=pl.BlockSpec((BE, D), blk),
        out_shape=jax.ShapeDtypeStruct((E, D), jnp.float32),
    )(a, b, w1a, w1b, b1, w2, b2)


BN = 1000


def _node_mlp_body(x_ref, g0_ref, g1_ref, w1x_ref, w1g_ref, b1_ref,
                   w2_ref, b2_ref, w3_ref, b3_ref, o_ref):
    g = g0_ref[...] + g1_ref[...]
    z = jnp.dot(x_ref[...], w1x_ref[...], preferred_element_type=jnp.float32)
    z = z + jnp.dot(g, w1g_ref[...], preferred_element_type=jnp.float32)
    h = _softplus(z + b1_ref[...])
    h = jnp.dot(h, w2_ref[...], preferred_element_type=jnp.float32)
    h = _softplus(h + b2_ref[...])
    h = jnp.dot(h, w3_ref[...], preferred_element_type=jnp.float32)
    o_ref[...] = h + b3_ref[...]


def _node_mlp(xf, g0, g1, w1x, w1g, b1, w2, b2, w3, b3):
    grid = (N // BN,)
    blk = lambda i: (i, 0)
    full = lambda i: (0, 0)
    return pl.pallas_call(
        _node_mlp_body,
        grid=grid,
        in_specs=[
            pl.BlockSpec((BN, D), blk),
            pl.BlockSpec((BN, D), blk),
            pl.BlockSpec((BN, D), blk),
            pl.BlockSpec((D, D), full),
            pl.BlockSpec((D, D), full),
            pl.BlockSpec((1, D), full),
            pl.BlockSpec((D, D), full),
            pl.BlockSpec((1, D), full),
            pl.BlockSpec((D, D), full),
            pl.BlockSpec((1, D), full),
        ],
        out_specs=pl.BlockSpec((BN, D), blk),
        out_shape=jax.ShapeDtypeStruct((N, D), jnp.float32),
    )(xf, g0, g1, w1x, w1g, b1, w2, b2, w3, b3)


def kernel(x, edge_idx, We1, be1, We2, be2, Wn1, bn1, Wn2, bn2, Wn3, bn3):
    xf = x.reshape(N, D)
    row = edge_idx[0].astype(jnp.int32)
    col = edge_idx[1].astype(jnp.int32)

    a, b = _sc_gather(xf, row, col)
    m = _edge_mlp(a, b, We1[:D].astype(jnp.bfloat16),
                  We1[D:].astype(jnp.bfloat16), be1.reshape(1, D),
                  We2.astype(jnp.bfloat16), be2.reshape(1, D))
    parts = _sc_scatter(m, row, jnp.zeros((N, D), jnp.float32))
    out = _node_mlp(xf, parts[:N], parts[N:], Wn1[:D], Wn1[D:],
                    bn1.reshape(1, D), Wn2, bn2.reshape(1, D),
                    Wn3, bn3.reshape(1, D))
    return out.reshape(1, N, D)


# speedup vs baseline: 4.0931x; 1.0314x over previous
import functools

import jax
import jax.numpy as jnp
from jax import lax
from jax.experimental import pallas as pl
from jax.experimental.pallas import tpu as pltpu
from jax.experimental.pallas import tpu_sc as plsc

N = 10000
E = 320000
D = 128

NC = 2
NS = 16
NW = NC * NS
NSPLIT = 2
EC = E // NSPLIT
EPW = EC // NW
GC = 40
NCHUNK = EPW // GC
NPAIR = (NCHUNK - 1) // 2
STRIPE = 632
STRIPE_LAST = N - (NS - 1) * STRIPE

assert EPW % GC == 0 and GC % 8 == 0 and NCHUNK % 2 == 1

_sc_mesh = plsc.VectorSubcoreMesh(core_axis_name="c", subcore_axis_name="s")


@functools.partial(
    pl.kernel,
    out_type=(
        jax.ShapeDtypeStruct((EC, D), jnp.float32),
        jax.ShapeDtypeStruct((EC, D), jnp.float32),
    ),
    mesh=_sc_mesh,
    scratch_types=[
        pltpu.VMEM((GC,), jnp.int32),
        pltpu.VMEM((GC,), jnp.int32),
        pltpu.VMEM((GC, D), jnp.float32),
        pltpu.VMEM((GC, D), jnp.float32),
        pltpu.VMEM((GC,), jnp.int32),
        pltpu.VMEM((GC,), jnp.int32),
        pltpu.VMEM((GC, D), jnp.float32),
        pltpu.VMEM((GC, D), jnp.float32),
        pltpu.SemaphoreType.DMA,
        pltpu.SemaphoreType.DMA,
        pltpu.SemaphoreType.DMA,
        pltpu.SemaphoreType.DMA,
    ],
)
def _sc_gather(xf_hbm, row_hbm, col_hbm, a_hbm, b_hbm,
               ridx0, cidx0, arows0, brows0,
               ridx1, cidx1, arows1, brows1,
               sem_a0, sem_b0, sem_a1, sem_b1):
    c = lax.axis_index("c")
    s = lax.axis_index("s")
    wid = s * NC + c
    base = wid * EPW

    def fire(chunk, ridx, cidx, arows, brows, sem_a, sem_b):
        off = base + chunk * GC
        pltpu.sync_copy(row_hbm.at[pl.ds(off, GC)], ridx)
        pltpu.sync_copy(col_hbm.at[pl.ds(off, GC)], cidx)
        pltpu.async_copy(xf_hbm.at[ridx], arows, sem_a)
        pltpu.async_copy(xf_hbm.at[cidx], brows, sem_b)

    def drain(chunk, ridx, cidx, arows, brows, sem_a, sem_b):
        off = base + chunk * GC
        pltpu.make_async_copy(xf_hbm.at[ridx], arows, sem_a).wait()
        pltpu.make_async_copy(xf_hbm.at[cidx], brows, sem_b).wait()
        pltpu.sync_copy(arows, a_hbm.at[pl.ds(off, GC)])
        pltpu.sync_copy(brows, b_hbm.at[pl.ds(off, GC)])

    fire(0, ridx0, cidx0, arows0, brows0, sem_a0, sem_b0)

    def step(j, carry):
        fire(2 * j + 1, ridx1, cidx1, arows1, brows1, sem_a1, sem_b1)
        drain(2 * j, ridx0, cidx0, arows0, brows0, sem_a0, sem_b0)
        fire(2 * j + 2, ridx0, cidx0, arows0, brows0, sem_a0, sem_b0)
        drain(2 * j + 1, ridx1, cidx1, arows1, brows1, sem_a1, sem_b1)
        return carry

    lax.fori_loop(0, NPAIR, step, 0)
    drain(NCHUNK - 1, ridx0, cidx0, arows0, brows0, sem_a0, sem_b0)


@functools.partial(
    pl.kernel,
    out_type=jax.ShapeDtypeStruct((2 * N, D), jnp.float32),
    mesh=_sc_mesh,
    scratch_types=[
        pltpu.VMEM((GC,), jnp.int32),
        pltpu.VMEM((GC, D), jnp.float32),
        pltpu.VMEM((GC,), jnp.int32),
        pltpu.VMEM((GC, D), jnp.float32),
        pltpu.VMEM_SHARED((N, D), jnp.float32),
        pltpu.SemaphoreType.DMA,
        pltpu.SemaphoreType.DMA,
        pltpu.SemaphoreType.DMA,
        pltpu.SemaphoreType.DMA,
    ],
)
def _sc_scatter(m_hbm, row_hbm, zeros_hbm, out_hbm,
                idx0, m0, idx1, m1, agg_sh,
                sem_i0, sem_m0, sem_i1, sem_m1):
    c = lax.axis_index("c")
    s = lax.axis_index("s")
    wid = s * NC + c
    base = wid * EPW

    @pl.when(s < NS - 1)
    def _():
        pltpu.sync_copy(zeros_hbm.at[pl.ds(s * STRIPE, STRIPE)],
                        agg_sh.at[pl.ds(s * STRIPE, STRIPE)])

    @pl.when(s == NS - 1)
    def _():
        pltpu.sync_copy(zeros_hbm.at[pl.ds(s * STRIPE, STRIPE_LAST)],
                        agg_sh.at[pl.ds(s * STRIPE, STRIPE_LAST)])

    plsc.subcore_barrier()

    def fire(chunk, idx, m, sem_i, sem_m):
        off = base + chunk * GC
        pltpu.async_copy(row_hbm.at[pl.ds(off, GC)], idx, sem_i)
        pltpu.async_copy(m_hbm.at[pl.ds(off, GC)], m, sem_m)

    def drain(chunk, idx, m, sem_i, sem_m):
        off = base + chunk * GC
        pltpu.make_async_copy(row_hbm.at[pl.ds(off, GC)], idx, sem_i).wait()
        pltpu.make_async_copy(m_hbm.at[pl.ds(off, GC)], m, sem_m).wait()
        pltpu.sync_copy(m, agg_sh.at[idx], add=True)

    fire(0, idx0, m0, sem_i0, sem_m0)

    def step(j, carry):
        fire(2 * j + 1, idx1, m1, sem_i1, sem_m1)
        drain(2 * j, idx0, m0, sem_i0, sem_m0)
        fire(2 * j + 2, idx0, m0, sem_i0, sem_m0)
        drain(2 * j + 1, idx1, m1, sem_i1, sem_m1)
        return carry

    lax.fori_loop(0, NPAIR, step, 0)
    drain(NCHUNK - 1, idx0, m0, sem_i0, sem_m0)

    plsc.subcore_barrier()

    @pl.when(s < NS - 1)
    def _():
        pltpu.sync_copy(agg_sh.at[pl.ds(s * STRIPE, STRIPE)],
                        out_hbm.at[pl.ds(c * N + s * STRIPE, STRIPE)])

    @pl.when(s == NS - 1)
    def _():
        pltpu.sync_copy(agg_sh.at[pl.ds(s * STRIPE, STRIPE_LAST)],
                        out_hbm.at[pl.ds(c * N + s * STRIPE, STRIPE_LAST)])


def _softplus(z):
    return jnp.maximum(z, 0.0) + jnp.log1p(jnp.exp(-jnp.abs(z)))


BE = 2000


def _edge_mlp_body(a_ref, b_ref, w1a_ref, w1b_ref, b1_ref, w2_ref, b2_ref,
                   o_ref):
    a = a_ref[...].astype(jnp.bfloat16)
    b = b_ref[...].astype(jnp.bfloat16)
    z = jnp.dot(a, w1a_ref[...], preferred_element_type=jnp.float32)
    z = z + jnp.dot(b, w1b_ref[...], preferred_element_type=jnp.float32)
    h = _softplus(z + b1_ref[...])
    h = jnp.dot(h.astype(jnp.bfloat16), w2_ref[...],
                preferred_element_type=jnp.float32)
    o_ref[...] = _softplus(h + b2_ref[...])


def _edge_mlp(a, b, w1a, w1b, b1, w2, b2):
    grid = (EC // BE,)
    blk = lambda i: (i, 0)
    full = lambda i: (0, 0)
    return pl.pallas_call(
        _edge_mlp_body,
        grid=grid,
        in_specs=[
            pl.BlockSpec((BE, D), blk),
            pl.BlockSpec((BE, D), blk),
            pl.BlockSpec((D, D), full),
            pl.BlockSpec((D, D), full),
            pl.BlockSpec((1, D), full),
            pl.BlockSpec((D, D), full),
            pl.BlockSpec((1, D), full),
        ],
        out_specs=pl.BlockSpec((BE, D), blk),
        out_shape=jax.ShapeDtypeStruct((EC, D), jnp.float32),
    )(a, b, w1a, w1b, b1, w2, b2)


BN = 1000


def _node_mlp_body(x_ref, g_refs, w1x_ref, w1g_ref, b1_ref,
                   w2_ref, b2_ref, w3_ref, b3_ref, o_ref):
    g = g_refs[0][...]
    for gr in g_refs[1:]:
        g = g + gr[...]
    z = jnp.dot(x_ref[...], w1x_ref[...], preferred_element_type=jnp.float32)
    z = z + jnp.dot(g, w1g_ref[...], preferred_element_type=jnp.float32)
    h = _softplus(z + b1_ref[...])
    h = jnp.dot(h, w2_ref[...], preferred_element_type=jnp.float32)
    h = _softplus(h + b2_ref[...])
    h = jnp.dot(h, w3_ref[...], preferred_element_type=jnp.float32)
    o_ref[...] = h + b3_ref[...]


def _node_mlp(xf, gs, w1x, w1g, b1, w2, b2, w3, b3):
    grid = (N // BN,)
    blk = lambda i: (i, 0)
    full = lambda i: (0, 0)

    def body(x_ref, *refs):
        ng = len(gs)
        _node_mlp_body(x_ref, refs[:ng], *refs[ng:])

    return pl.pallas_call(
        body,
        grid=grid,
        in_specs=[pl.BlockSpec((BN, D), blk)]
        + [pl.BlockSpec((BN, D), blk) for _ in gs]
        + [
            pl.BlockSpec((D, D), full),
            pl.BlockSpec((D, D), full),
            pl.BlockSpec((1, D), full),
            pl.BlockSpec((D, D), full),
            pl.BlockSpec((1, D), full),
            pl.BlockSpec((D, D), full),
            pl.BlockSpec((1, D), full),
        ],
        out_specs=pl.BlockSpec((BN, D), blk),
        out_shape=jax.ShapeDtypeStruct((N, D), jnp.float32),
    )(xf, *gs, w1x, w1g, b1, w2, b2, w3, b3)


def kernel(x, edge_idx, We1, be1, We2, be2, Wn1, bn1, Wn2, bn2, Wn3, bn3):
    xf = x.reshape(N, D)
    row = edge_idx[0].astype(jnp.int32)
    col = edge_idx[1].astype(jnp.int32)
    zeros = jnp.zeros((N, D), jnp.float32)

    w1a = We1[:D].astype(jnp.bfloat16)
    w1b = We1[D:].astype(jnp.bfloat16)
    w2 = We2.astype(jnp.bfloat16)

    parts = []
    for k in range(NSPLIT):
        sl = slice(k * EC, (k + 1) * EC)
        a, b = _sc_gather(xf, row[sl], col[sl])
        m = _edge_mlp(a, b, w1a, w1b, be1.reshape(1, D), w2,
                      be2.reshape(1, D))
        p = _sc_scatter(m, row[sl], zeros)
        parts.append(p[:N])
        parts.append(p[N:])

    out = _node_mlp(xf, parts, Wn1[:D], Wn1[D:],
                    bn1.reshape(1, D), Wn2, bn2.reshape(1, D),
                    Wn3, bn3.reshape(1, D))
    return out.reshape(1, N, D)


# speedup vs baseline: 4.8601x; 1.1874x over previous
import functools

import jax
import jax.numpy as jnp
from jax import lax
from jax.experimental import pallas as pl
from jax.experimental.pallas import tpu as pltpu
from jax.experimental.pallas import tpu_sc as plsc

N = 10000
E = 320000
D = 128

NC = 2
NS = 16
NW = NC * NS
NSPLIT = 2
EC = E // NSPLIT
EPW = EC // NW
GC = 128
NFULL = EPW // GC
TAIL = EPW - NFULL * GC
NPAIR = (NFULL - 1) // 2
STRIPE = 632
STRIPE_LAST = N - (NS - 1) * STRIPE

assert NFULL % 2 == 1 and TAIL % 8 == 0 and TAIL > 0

_sc_mesh = plsc.VectorSubcoreMesh(core_axis_name="c", subcore_axis_name="s")


@functools.partial(
    pl.kernel,
    out_type=(
        jax.ShapeDtypeStruct((EC, D), jnp.float32),
        jax.ShapeDtypeStruct((EC, D), jnp.float32),
    ),
    mesh=_sc_mesh,
    scratch_types=[
        pltpu.VMEM((GC,), jnp.int32),
        pltpu.VMEM((GC,), jnp.int32),
        pltpu.VMEM((GC, D), jnp.float32),
        pltpu.VMEM((GC, D), jnp.float32),
        pltpu.VMEM((GC,), jnp.int32),
        pltpu.VMEM((GC,), jnp.int32),
        pltpu.VMEM((GC, D), jnp.float32),
        pltpu.VMEM((GC, D), jnp.float32),
        pltpu.VMEM((TAIL,), jnp.int32),
        pltpu.VMEM((TAIL,), jnp.int32),
        pltpu.VMEM((TAIL, D), jnp.float32),
        pltpu.VMEM((TAIL, D), jnp.float32),
        pltpu.SemaphoreType.DMA,
        pltpu.SemaphoreType.DMA,
        pltpu.SemaphoreType.DMA,
        pltpu.SemaphoreType.DMA,
    ],
)
def _sc_gather(xf_hbm, row_hbm, col_hbm, a_hbm, b_hbm,
               ridx0, cidx0, arows0, brows0,
               ridx1, cidx1, arows1, brows1,
               ridxt, cidxt, arowst, browst,
               sem_a0, sem_b0, sem_a1, sem_b1):
    c = lax.axis_index("c")
    s = lax.axis_index("s")
    wid = s * NC + c
    base = wid * EPW

    def fire(chunk, n, ridx, cidx, arows, brows, sem_a, sem_b):
        off = base + chunk * GC
        pltpu.sync_copy(row_hbm.at[pl.ds(off, n)], ridx)
        pltpu.sync_copy(col_hbm.at[pl.ds(off, n)], cidx)
        pltpu.async_copy(xf_hbm.at[ridx], arows, sem_a)
        pltpu.async_copy(xf_hbm.at[cidx], brows, sem_b)

    def drain(chunk, n, ridx, cidx, arows, brows, sem_a, sem_b):
        off = base + chunk * GC
        pltpu.make_async_copy(xf_hbm.at[ridx], arows, sem_a).wait()
        pltpu.make_async_copy(xf_hbm.at[cidx], brows, sem_b).wait()
        pltpu.sync_copy(arows, a_hbm.at[pl.ds(off, n)])
        pltpu.sync_copy(brows, b_hbm.at[pl.ds(off, n)])

    fire(0, GC, ridx0, cidx0, arows0, brows0, sem_a0, sem_b0)

    def step(j, carry):
        fire(2 * j + 1, GC, ridx1, cidx1, arows1, brows1, sem_a1, sem_b1)
        drain(2 * j, GC, ridx0, cidx0, arows0, brows0, sem_a0, sem_b0)
        fire(2 * j + 2, GC, ridx0, cidx0, arows0, brows0, sem_a0, sem_b0)
        drain(2 * j + 1, GC, ridx1, cidx1, arows1, brows1, sem_a1, sem_b1)
        return carry

    lax.fori_loop(0, NPAIR, step, 0)
    fire(NFULL, TAIL, ridxt, cidxt, arowst, browst, sem_a1, sem_b1)
    drain(NFULL - 1, GC, ridx0, cidx0, arows0, brows0, sem_a0, sem_b0)
    drain(NFULL, TAIL, ridxt, cidxt, arowst, browst, sem_a1, sem_b1)


@functools.partial(
    pl.kernel,
    out_type=jax.ShapeDtypeStruct((2 * N, D), jnp.float32),
    mesh=_sc_mesh,
    scratch_types=[
        pltpu.VMEM((GC,), jnp.int32),
        pltpu.VMEM((GC, D), jnp.float32),
        pltpu.VMEM((GC,), jnp.int32),
        pltpu.VMEM((GC, D), jnp.float32),
        pltpu.VMEM((TAIL,), jnp.int32),
        pltpu.VMEM((TAIL, D), jnp.float32),
        pltpu.VMEM_SHARED((N, D), jnp.float32),
        pltpu.SemaphoreType.DMA,
        pltpu.SemaphoreType.DMA,
        pltpu.SemaphoreType.DMA,
        pltpu.SemaphoreType.DMA,
    ],
)
def _sc_scatter(m_hbm, row_hbm, init_hbm, out_hbm,
                idx0, m0, idx1, m1, idxt, mt, agg_sh,
                sem_i0, sem_m0, sem_i1, sem_m1):
    c = lax.axis_index("c")
    s = lax.axis_index("s")
    wid = s * NC + c
    base = wid * EPW

    @pl.when(s < NS - 1)
    def _():
        pltpu.sync_copy(init_hbm.at[pl.ds(c * N + s * STRIPE, STRIPE)],
                        agg_sh.at[pl.ds(s * STRIPE, STRIPE)])

    @pl.when(s == NS - 1)
    def _():
        pltpu.sync_copy(init_hbm.at[pl.ds(c * N + s * STRIPE, STRIPE_LAST)],
                        agg_sh.at[pl.ds(s * STRIPE, STRIPE_LAST)])

    plsc.subcore_barrier()

    def fire(chunk, n, idx, m, sem_i, sem_m):
        off = base + chunk * GC
        pltpu.async_copy(row_hbm.at[pl.ds(off, n)], idx, sem_i)
        pltpu.async_copy(m_hbm.at[pl.ds(off, n)], m, sem_m)

    def drain(chunk, n, idx, m, sem_i, sem_m):
        off = base + chunk * GC
        pltpu.make_async_copy(row_hbm.at[pl.ds(off, n)], idx, sem_i).wait()
        pltpu.make_async_copy(m_hbm.at[pl.ds(off, n)], m, sem_m).wait()
        pltpu.sync_copy(m, agg_sh.at[idx], add=True)

    fire(0, GC, idx0, m0, sem_i0, sem_m0)

    def step(j, carry):
        fire(2 * j + 1, GC, idx1, m1, sem_i1, sem_m1)
        drain(2 * j, GC, idx0, m0, sem_i0, sem_m0)
        fire(2 * j + 2, GC, idx0, m0, sem_i0, sem_m0)
        drain(2 * j + 1, GC, idx1, m1, sem_i1, sem_m1)
        return carry

    lax.fori_loop(0, NPAIR, step, 0)
    fire(NFULL, TAIL, idxt, mt, sem_i1, sem_m1)
    drain(NFULL - 1, GC, idx0, m0, sem_i0, sem_m0)
    drain(NFULL, TAIL, idxt, mt, sem_i1, sem_m1)

    plsc.subcore_barrier()

    @pl.when(s < NS - 1)
    def _():
        pltpu.sync_copy(agg_sh.at[pl.ds(s * STRIPE, STRIPE)],
                        out_hbm.at[pl.ds(c * N + s * STRIPE, STRIPE)])

    @pl.when(s == NS - 1)
    def _():
        pltpu.sync_copy(agg_sh.at[pl.ds(s * STRIPE, STRIPE_LAST)],
                        out_hbm.at[pl.ds(c * N + s * STRIPE, STRIPE_LAST)])


def _softplus(z):
    return jnp.maximum(z, 0.0) + jnp.log1p(jnp.exp(-jnp.abs(z)))


BE = 2000


def _edge_mlp_body(a_ref, b_ref, w1a_ref, w1b_ref, b1_ref, w2_ref, b2_ref,
                   o_ref):
    a = a_ref[...].astype(jnp.bfloat16)
    b = b_ref[...].astype(jnp.bfloat16)
    z = jnp.dot(a, w1a_ref[...], preferred_element_type=jnp.float32)
    z = z + jnp.dot(b, w1b_ref[...], preferred_element_type=jnp.float32)
    h = _softplus(z + b1_ref[...])
    h = jnp.dot(h.astype(jnp.bfloat16), w2_ref[...],
                preferred_element_type=jnp.float32)
    o_ref[...] = _softplus(h + b2_ref[...])


def _edge_mlp(a, b, w1a, w1b, b1, w2, b2):
    grid = (EC // BE,)
    blk = lambda i: (i, 0)
    full = lambda i: (0, 0)
    return pl.pallas_call(
        _edge_mlp_body,
        grid=grid,
        in_specs=[
            pl.BlockSpec((BE, D), blk),
            pl.BlockSpec((BE, D), blk),
            pl.BlockSpec((D, D), full),
            pl.BlockSpec((D, D), full),
            pl.BlockSpec((1, D), full),
            pl.BlockSpec((D, D), full),
            pl.BlockSpec((1, D), full),
        ],
        out_specs=pl.BlockSpec((BE, D), blk),
        out_shape=jax.ShapeDtypeStruct((EC, D), jnp.float32),
    )(a, b, w1a, w1b, b1, w2, b2)


BN = 1000


def _node_mlp_body(x_ref, g0_ref, g1_ref, w1x_ref, w1g_ref, b1_ref,
                   w2_ref, b2_ref, w3_ref, b3_ref, o_ref):
    g = g0_ref[...] + g1_ref[...]
    z = jnp.dot(x_ref[...], w1x_ref[...], preferred_element_type=jnp.float32)
    z = z + jnp.dot(g, w1g_ref[...], preferred_element_type=jnp.float32)
    h = _softplus(z + b1_ref[...])
    h = jnp.dot(h, w2_ref[...], preferred_element_type=jnp.float32)
    h = _softplus(h + b2_ref[...])
    h = jnp.dot(h, w3_ref[...], preferred_element_type=jnp.float32)
    o_ref[...] = h + b3_ref[...]


def _node_mlp(xf, g0, g1, w1x, w1g, b1, w2, b2, w3, b3):
    grid = (N // BN,)
    blk = lambda i: (i, 0)
    full = lambda i: (0, 0)
    return pl.pallas_call(
        _node_mlp_body,
        grid=grid,
        in_specs=[
            pl.BlockSpec((BN, D), blk),
            pl.BlockSpec((BN, D), blk),
            pl.BlockSpec((BN, D), blk),
            pl.BlockSpec((D, D), full),
            pl.BlockSpec((D, D), full),
            pl.BlockSpec((1, D), full),
            pl.BlockSpec((D, D), full),
            pl.BlockSpec((1, D), full),
            pl.BlockSpec((D, D), full),
            pl.BlockSpec((1, D), full),
        ],
        out_specs=pl.BlockSpec((BN, D), blk),
        out_shape=jax.ShapeDtypeStruct((N, D), jnp.float32),
    )(xf, g0, g1, w1x, w1g, b1, w2, b2, w3, b3)


def kernel(x, edge_idx, We1, be1, We2, be2, Wn1, bn1, Wn2, bn2, Wn3, bn3):
    xf = x.reshape(N, D)
    row = edge_idx[0].astype(jnp.int32)
    col = edge_idx[1].astype(jnp.int32)

    w1a = We1[:D].astype(jnp.bfloat16)
    w1b = We1[D:].astype(jnp.bfloat16)
    w2 = We2.astype(jnp.bfloat16)

    acc = jnp.zeros((2 * N, D), jnp.float32)
    for k in range(NSPLIT):
        sl = slice(k * EC, (k + 1) * EC)
        a, b = _sc_gather(xf, row[sl], col[sl])
        m = _edge_mlp(a, b, w1a, w1b, be1.reshape(1, D), w2,
                      be2.reshape(1, D))
        acc = _sc_scatter(m, row[sl], acc)

    out = _node_mlp(xf, acc[:N], acc[N:], Wn1[:D], Wn1[D:],
                    bn1.reshape(1, D), Wn2, bn2.reshape(1, D),
                    Wn3, bn3.reshape(1, D))
    return out.reshape(1, N, D)


# speedup vs baseline: 5.7282x; 1.1786x over previous
import functools

import jax
import jax.numpy as jnp
from jax import lax
from jax.experimental import pallas as pl
from jax.experimental.pallas import tpu as pltpu
from jax.experimental.pallas import tpu_sc as plsc

N = 10000
E = 320000
D = 128

NC = 2
NS = 16
NW = NC * NS
NSPLIT = 2
EC = E // NSPLIT
EPW = EC // NW
GC = 128
NFULL = EPW // GC
TAIL = EPW - NFULL * GC
NPAIR = (NFULL - 1) // 2
STRIPE = 632
STRIPE_LAST = N - (NS - 1) * STRIPE

assert NFULL % 2 == 1 and TAIL % 8 == 0 and TAIL > 0

_sc_mesh = plsc.VectorSubcoreMesh(core_axis_name="c", subcore_axis_name="s")


@functools.partial(
    pl.kernel,
    out_type=jax.ShapeDtypeStruct((EC, D), jnp.float32),
    mesh=_sc_mesh,
    scratch_types=[
        pltpu.VMEM((GC,), jnp.int32),
        pltpu.VMEM((GC,), jnp.int32),
        pltpu.VMEM((GC, D), jnp.float32),
        pltpu.VMEM((GC, D), jnp.float32),
        pltpu.VMEM((GC,), jnp.int32),
        pltpu.VMEM((GC,), jnp.int32),
        pltpu.VMEM((GC, D), jnp.float32),
        pltpu.VMEM((GC, D), jnp.float32),
        pltpu.VMEM((TAIL,), jnp.int32),
        pltpu.VMEM((TAIL,), jnp.int32),
        pltpu.VMEM((TAIL, D), jnp.float32),
        pltpu.VMEM((TAIL, D), jnp.float32),
        pltpu.VMEM((GC,), jnp.int32),
        pltpu.VMEM((GC,), jnp.int32),
        pltpu.VMEM_SHARED((NS * 2 * GC, D), jnp.float32),
        pltpu.SemaphoreType.DMA,
        pltpu.SemaphoreType.DMA,
        pltpu.SemaphoreType.DMA,
        pltpu.SemaphoreType.DMA,
    ],
)
def _sc_gather_add(p_hbm, q_hbm, row_hbm, col_hbm, z_hbm,
                   ridx0, cidx0, arows0, brows0,
                   ridx1, cidx1, arows1, brows1,
                   ridxt, cidxt, arowst, browst, ident0, ident1, zsp,
                   sem_a0, sem_b0, sem_a1, sem_b1):
    c = lax.axis_index("c")
    s = lax.axis_index("s")
    wid = s * NC + c
    base = wid * EPW

    for k in range(GC // 16):
        off16 = lax.iota(jnp.int32, 16) + (16 * k) + s * (2 * GC)
        ident0[pl.ds(16 * k, 16)] = off16
        ident1[pl.ds(16 * k, 16)] = off16 + GC

    def fire(chunk, n, ridx, cidx, arows, brows, sem_a, sem_b):
        off = base + chunk * GC
        pltpu.sync_copy(row_hbm.at[pl.ds(off, n)], ridx)
        pltpu.sync_copy(col_hbm.at[pl.ds(off, n)], cidx)
        pltpu.async_copy(p_hbm.at[ridx], arows, sem_a)
        pltpu.async_copy(q_hbm.at[cidx], brows, sem_b)

    def drain(chunk, par, ridx, cidx, arows, brows, ident, sem_a, sem_b):
        off = base + chunk * GC
        zoff = s * (2 * GC) + par * GC
        pltpu.make_async_copy(p_hbm.at[ridx], arows, sem_a).wait()
        pltpu.make_async_copy(q_hbm.at[cidx], brows, sem_b).wait()
        pltpu.sync_copy(arows, zsp.at[pl.ds(zoff, GC)])
        pltpu.sync_copy(brows, zsp.at[ident], add=True)
        pltpu.sync_copy(zsp.at[pl.ds(zoff, GC)], z_hbm.at[pl.ds(off, GC)])

    fire(0, GC, ridx0, cidx0, arows0, brows0, sem_a0, sem_b0)

    def step(j, carry):
        fire(2 * j + 1, GC, ridx1, cidx1, arows1, brows1, sem_a1, sem_b1)
        drain(2 * j, 0, ridx0, cidx0, arows0, brows0, ident0,
              sem_a0, sem_b0)
        fire(2 * j + 2, GC, ridx0, cidx0, arows0, brows0, sem_a0, sem_b0)
        drain(2 * j + 1, 1, ridx1, cidx1, arows1, brows1, ident1,
              sem_a1, sem_b1)
        return carry

    lax.fori_loop(0, NPAIR, step, 0)
    fire(NFULL, TAIL, ridxt, cidxt, arowst, browst, sem_a1, sem_b1)
    drain(NFULL - 1, 0, ridx0, cidx0, arows0, brows0, ident0,
          sem_a0, sem_b0)
    off_t = base + NFULL * GC
    pltpu.make_async_copy(p_hbm.at[ridxt], arowst, sem_a1).wait()
    pltpu.make_async_copy(q_hbm.at[cidxt], browst, sem_b1).wait()
    for r in range(TAIL):
        for k in range(D // 16):
            sl = pl.ds(16 * k, 16)
            arowst[r, sl] = arowst[r, sl] + browst[r, sl]
    pltpu.sync_copy(arowst, z_hbm.at[pl.ds(off_t, TAIL)])


@functools.partial(
    pl.kernel,
    out_type=jax.ShapeDtypeStruct((2 * N, D), jnp.float32),
    mesh=_sc_mesh,
    scratch_types=[
        pltpu.VMEM((GC,), jnp.int32),
        pltpu.VMEM((GC, D), jnp.float32),
        pltpu.VMEM((GC,), jnp.int32),
        pltpu.VMEM((GC, D), jnp.float32),
        pltpu.VMEM((TAIL,), jnp.int32),
        pltpu.VMEM((TAIL, D), jnp.float32),
        pltpu.VMEM_SHARED((N, D), jnp.float32),
        pltpu.SemaphoreType.DMA,
        pltpu.SemaphoreType.DMA,
        pltpu.SemaphoreType.DMA,
        pltpu.SemaphoreType.DMA,
    ],
)
def _sc_scatter(m_hbm, row_hbm, init_hbm, out_hbm,
                idx0, m0, idx1, m1, idxt, mt, agg_sh,
                sem_i0, sem_m0, sem_i1, sem_m1):
    c = lax.axis_index("c")
    s = lax.axis_index("s")
    wid = s * NC + c
    base = wid * EPW

    @pl.when(s < NS - 1)
    def _():
        pltpu.sync_copy(init_hbm.at[pl.ds(c * N + s * STRIPE, STRIPE)],
                        agg_sh.at[pl.ds(s * STRIPE, STRIPE)])

    @pl.when(s == NS - 1)
    def _():
        pltpu.sync_copy(init_hbm.at[pl.ds(c * N + s * STRIPE, STRIPE_LAST)],
                        agg_sh.at[pl.ds(s * STRIPE, STRIPE_LAST)])

    plsc.subcore_barrier()

    def fire(chunk, n, idx, m, sem_i, sem_m):
        off = base + chunk * GC
        pltpu.async_copy(row_hbm.at[pl.ds(off, n)], idx, sem_i)
        pltpu.async_copy(m_hbm.at[pl.ds(off, n)], m, sem_m)

    def drain(chunk, n, idx, m, sem_i, sem_m):
        off = base + chunk * GC
        pltpu.make_async_copy(row_hbm.at[pl.ds(off, n)], idx, sem_i).wait()
        pltpu.make_async_copy(m_hbm.at[pl.ds(off, n)], m, sem_m).wait()
        pltpu.sync_copy(m, agg_sh.at[idx], add=True)

    fire(0, GC, idx0, m0, sem_i0, sem_m0)

    def step(j, carry):
        fire(2 * j + 1, GC, idx1, m1, sem_i1, sem_m1)
        drain(2 * j, GC, idx0, m0, sem_i0, sem_m0)
        fire(2 * j + 2, GC, idx0, m0, sem_i0, sem_m0)
        drain(2 * j + 1, GC, idx1, m1, sem_i1, sem_m1)
        return carry

    lax.fori_loop(0, NPAIR, step, 0)
    fire(NFULL, TAIL, idxt, mt, sem_i1, sem_m1)
    drain(NFULL - 1, GC, idx0, m0, sem_i0, sem_m0)
    drain(NFULL, TAIL, idxt, mt, sem_i1, sem_m1)

    plsc.subcore_barrier()

    @pl.when(s < NS - 1)
    def _():
        pltpu.sync_copy(agg_sh.at[pl.ds(s * STRIPE, STRIPE)],
                        out_hbm.at[pl.ds(c * N + s * STRIPE, STRIPE)])

    @pl.when(s == NS - 1)
    def _():
        pltpu.sync_copy(agg_sh.at[pl.ds(s * STRIPE, STRIPE_LAST)],
                        out_hbm.at[pl.ds(c * N + s * STRIPE, STRIPE_LAST)])


def _softplus(z):
    return jnp.maximum(z, 0.0) + jnp.log1p(jnp.exp(-jnp.abs(z)))


BE = 2000
BP = 1000


def _proj_body(x_ref, w1a_ref, w1b_ref, b1_ref, p_ref, q_ref):
    xb = x_ref[...].astype(jnp.bfloat16)
    p_ref[...] = jnp.dot(xb, w1a_ref[...],
                         preferred_element_type=jnp.float32) + b1_ref[...]
    q_ref[...] = jnp.dot(xb, w1b_ref[...],
                         preferred_element_type=jnp.float32)


def _proj(xf, w1a, w1b, b1):
    grid = (N // BP,)
    blk = lambda i: (i, 0)
    full = lambda i: (0, 0)
    return pl.pallas_call(
        _proj_body,
        grid=grid,
        in_specs=[
            pl.BlockSpec((BP, D), blk),
            pl.BlockSpec((D, D), full),
            pl.BlockSpec((D, D), full),
            pl.BlockSpec((1, D), full),
        ],
        out_specs=(pl.BlockSpec((BP, D), blk), pl.BlockSpec((BP, D), blk)),
        out_shape=(jax.ShapeDtypeStruct((N, D), jnp.float32),
                   jax.ShapeDtypeStruct((N, D), jnp.float32)),
    )(xf, w1a, w1b, b1)


def _edge_mlp_body(z_ref, w2_ref, b2_ref, o_ref):
    h = _softplus(z_ref[...])
    h = jnp.dot(h.astype(jnp.bfloat16), w2_ref[...],
                preferred_element_type=jnp.float32)
    o_ref[...] = _softplus(h + b2_ref[...])


def _edge_mlp(z, w2, b2):
    grid = (EC // BE,)
    blk = lambda i: (i, 0)
    full = lambda i: (0, 0)
    return pl.pallas_call(
        _edge_mlp_body,
        grid=grid,
        in_specs=[
            pl.BlockSpec((BE, D), blk),
            pl.BlockSpec((D, D), full),
            pl.BlockSpec((1, D), full),
        ],
        out_specs=pl.BlockSpec((BE, D), blk),
        out_shape=jax.ShapeDtypeStruct((EC, D), jnp.float32),
    )(z, w2, b2)


BN = 1000


def _node_mlp_body(x_ref, g0_ref, g1_ref, w1x_ref, w1g_ref, b1_ref,
                   w2_ref, b2_ref, w3_ref, b3_ref, o_ref):
    g = g0_ref[...] + g1_ref[...]
    z = jnp.dot(x_ref[...], w1x_ref[...], preferred_element_type=jnp.float32)
    z = z + jnp.dot(g, w1g_ref[...], preferred_element_type=jnp.float32)
    h = _softplus(z + b1_ref[...])
    h = jnp.dot(h, w2_ref[...], preferred_element_type=jnp.float32)
    h = _softplus(h + b2_ref[...])
    h = jnp.dot(h, w3_ref[...], preferred_element_type=jnp.float32)
    o_ref[...] = h + b3_ref[...]


def _node_mlp(xf, g0, g1, w1x, w1g, b1, w2, b2, w3, b3):
    grid = (N // BN,)
    blk = lambda i: (i, 0)
    full = lambda i: (0, 0)
    return pl.pallas_call(
        _node_mlp_body,
        grid=grid,
        in_specs=[
            pl.BlockSpec((BN, D), blk),
            pl.BlockSpec((BN, D), blk),
            pl.BlockSpec((BN, D), blk),
            pl.BlockSpec((D, D), full),
            pl.BlockSpec((D, D), full),
            pl.BlockSpec((1, D), full),
            pl.BlockSpec((D, D), full),
            pl.BlockSpec((1, D), full),
            pl.BlockSpec((D, D), full),
            pl.BlockSpec((1, D), full),
        ],
        out_specs=pl.BlockSpec((BN, D), blk),
        out_shape=jax.ShapeDtypeStruct((N, D), jnp.float32),
    )(xf, g0, g1, w1x, w1g, b1, w2, b2, w3, b3)


def kernel(x, edge_idx, We1, be1, We2, be2, Wn1, bn1, Wn2, bn2, Wn3, bn3):
    xf = x.reshape(N, D)
    row = edge_idx[0].astype(jnp.int32)
    col = edge_idx[1].astype(jnp.int32)

    w1a = We1[:D].astype(jnp.bfloat16)
    w1b = We1[D:].astype(jnp.bfloat16)
    w2 = We2.astype(jnp.bfloat16)

    p, q = _proj(xf, w1a, w1b, be1.reshape(1, D))

    acc = jnp.zeros((2 * N, D), jnp.float32)
    for k in range(NSPLIT):
        sl = slice(k * EC, (k + 1) * EC)
        z = _sc_gather_add(p, q, row[sl], col[sl])
        m = _edge_mlp(z, w2, be2.reshape(1, D))
        acc = _sc_scatter(m, row[sl], acc)

    out = _node_mlp(xf, acc[:N], acc[N:], Wn1[:D], Wn1[D:],
                    bn1.reshape(1, D), Wn2, bn2.reshape(1, D),
                    Wn3, bn3.reshape(1, D))
    return out.reshape(1, N, D)


# speedup vs baseline: 6.0215x; 1.0512x over previous
import functools

import jax
import jax.numpy as jnp
from jax import lax
from jax.experimental import pallas as pl
from jax.experimental.pallas import tpu as pltpu
from jax.experimental.pallas import tpu_sc as plsc

N = 10000
E = 320000
D = 128

NC = 2
NS = 16
NW = NC * NS
NSPLIT = 2
EC = E // NSPLIT
EPW = EC // NW
GC = 128
NFULL = EPW // GC
TAIL = EPW - NFULL * GC
NPAIR = (NFULL - 1) // 2
STRIPE = 632
STRIPE_LAST = N - (NS - 1) * STRIPE

assert NFULL % 2 == 1 and TAIL % 8 == 0 and TAIL > 0

_sc_mesh = plsc.VectorSubcoreMesh(core_axis_name="c", subcore_axis_name="s")


@functools.partial(
    pl.kernel,
    out_type=jax.ShapeDtypeStruct((EC, D), jnp.float32),
    mesh=_sc_mesh,
    scratch_types=[
        pltpu.VMEM((GC,), jnp.int32),
        pltpu.VMEM((GC,), jnp.int32),
        pltpu.VMEM((GC, D), jnp.float32),
        pltpu.VMEM((GC, D), jnp.float32),
        pltpu.VMEM((GC,), jnp.int32),
        pltpu.VMEM((GC,), jnp.int32),
        pltpu.VMEM((GC, D), jnp.float32),
        pltpu.VMEM((GC, D), jnp.float32),
        pltpu.VMEM((TAIL,), jnp.int32),
        pltpu.VMEM((TAIL,), jnp.int32),
        pltpu.VMEM((TAIL, D), jnp.float32),
        pltpu.VMEM((TAIL, D), jnp.float32),
        pltpu.VMEM((GC,), jnp.int32),
        pltpu.VMEM((GC,), jnp.int32),
        pltpu.VMEM_SHARED((NS * 2 * GC, D), jnp.float32),
        pltpu.SemaphoreType.DMA,
        pltpu.SemaphoreType.DMA,
        pltpu.SemaphoreType.DMA,
        pltpu.SemaphoreType.DMA,
        pltpu.SemaphoreType.DMA,
        pltpu.SemaphoreType.DMA,
    ],
)
def _sc_gather_add(p_hbm, q_hbm, row_hbm, col_hbm, z_hbm,
                   ridx0, cidx0, arows0, brows0,
                   ridx1, cidx1, arows1, brows1,
                   ridxt, cidxt, arowst, browst, ident0, ident1, zsp,
                   sem_a0, sem_b0, sem_a1, sem_b1, sem_w0, sem_w1):
    c = lax.axis_index("c")
    s = lax.axis_index("s")
    wid = s * NC + c
    base = wid * EPW

    for k in range(GC // 16):
        off16 = lax.iota(jnp.int32, 16) + (16 * k) + s * (2 * GC)
        ident0[pl.ds(16 * k, 16)] = off16
        ident1[pl.ds(16 * k, 16)] = off16 + GC

    def fire(chunk, n, ridx, cidx, arows, brows, sem_a, sem_b):
        off = base + chunk * GC
        pltpu.sync_copy(row_hbm.at[pl.ds(off, n)], ridx)
        pltpu.sync_copy(col_hbm.at[pl.ds(off, n)], cidx)
        pltpu.async_copy(p_hbm.at[ridx], arows, sem_a)
        pltpu.async_copy(q_hbm.at[cidx], brows, sem_b)

    def drain(chunk, par, ridx, cidx, arows, brows, ident, sem_a, sem_b,
              sem_w):
        off = base + chunk * GC
        zoff = s * (2 * GC) + par * GC
        pltpu.make_async_copy(p_hbm.at[ridx], arows, sem_a).wait()
        pltpu.make_async_copy(q_hbm.at[cidx], brows, sem_b).wait()

        def _wait_wb():
            pltpu.make_async_copy(zsp.at[pl.ds(zoff, GC)],
                                  z_hbm.at[pl.ds(off, GC)], sem_w).wait()

        if isinstance(chunk, int):
            if chunk >= 2:
                _wait_wb()
        else:
            pl.when(chunk >= 2)(_wait_wb)

        pltpu.sync_copy(arows, zsp.at[pl.ds(zoff, GC)])
        pltpu.sync_copy(brows, zsp.at[ident], add=True)
        pltpu.async_copy(zsp.at[pl.ds(zoff, GC)], z_hbm.at[pl.ds(off, GC)],
                         sem_w)

    fire(0, GC, ridx0, cidx0, arows0, brows0, sem_a0, sem_b0)

    def step(j, carry):
        fire(2 * j + 1, GC, ridx1, cidx1, arows1, brows1, sem_a1, sem_b1)
        drain(2 * j, 0, ridx0, cidx0, arows0, brows0, ident0,
              sem_a0, sem_b0, sem_w0)
        fire(2 * j + 2, GC, ridx0, cidx0, arows0, brows0, sem_a0, sem_b0)
        drain(2 * j + 1, 1, ridx1, cidx1, arows1, brows1, ident1,
              sem_a1, sem_b1, sem_w1)
        return carry

    lax.fori_loop(0, NPAIR, step, 0)
    fire(NFULL, TAIL, ridxt, cidxt, arowst, browst, sem_a1, sem_b1)
    drain(NFULL - 1, 0, ridx0, cidx0, arows0, brows0, ident0,
          sem_a0, sem_b0, sem_w0)
    off_t = base + NFULL * GC
    pltpu.make_async_copy(p_hbm.at[ridxt], arowst, sem_a1).wait()
    pltpu.make_async_copy(q_hbm.at[cidxt], browst, sem_b1).wait()
    for r in range(TAIL):
        for k in range(D // 16):
            sl = pl.ds(16 * k, 16)
            arowst[r, sl] = arowst[r, sl] + browst[r, sl]
    pltpu.sync_copy(arowst, z_hbm.at[pl.ds(off_t, TAIL)])
    pltpu.make_async_copy(zsp.at[pl.ds(s * (2 * GC), GC)],
                          z_hbm.at[pl.ds(base, GC)], sem_w0).wait()
    pltpu.make_async_copy(zsp.at[pl.ds(s * (2 * GC) + GC, GC)],
                          z_hbm.at[pl.ds(base, GC)], sem_w1).wait()


@functools.partial(
    pl.kernel,
    out_type=jax.ShapeDtypeStruct((2 * N, D), jnp.float32),
    mesh=_sc_mesh,
    scratch_types=[
        pltpu.VMEM((GC,), jnp.int32),
        pltpu.VMEM((GC, D), jnp.float32),
        pltpu.VMEM((GC,), jnp.int32),
        pltpu.VMEM((GC, D), jnp.float32),
        pltpu.VMEM((TAIL,), jnp.int32),
        pltpu.VMEM((TAIL, D), jnp.float32),
        pltpu.VMEM_SHARED((N, D), jnp.float32),
        pltpu.SemaphoreType.DMA,
        pltpu.SemaphoreType.DMA,
        pltpu.SemaphoreType.DMA,
        pltpu.SemaphoreType.DMA,
        pltpu.SemaphoreType.DMA,
        pltpu.SemaphoreType.DMA,
    ],
)
def _sc_scatter(m_hbm, row_hbm, init_hbm, out_hbm,
                idx0, m0, idx1, m1, idxt, mt, agg_sh,
                sem_i0, sem_m0, sem_i1, sem_m1, sem_s0, sem_s1):
    c = lax.axis_index("c")
    s = lax.axis_index("s")
    wid = s * NC + c
    base = wid * EPW

    @pl.when(s < NS - 1)
    def _():
        pltpu.sync_copy(init_hbm.at[pl.ds(c * N + s * STRIPE, STRIPE)],
                        agg_sh.at[pl.ds(s * STRIPE, STRIPE)])

    @pl.when(s == NS - 1)
    def _():
        pltpu.sync_copy(init_hbm.at[pl.ds(c * N + s * STRIPE, STRIPE_LAST)],
                        agg_sh.at[pl.ds(s * STRIPE, STRIPE_LAST)])

    plsc.subcore_barrier()

    def fire(chunk, n, idx, m, sem_i, sem_m, sem_s):
        off = base + chunk * GC

        def _wait_s():
            pltpu.make_async_copy(m, agg_sh.at[idx], sem_s).wait()

        if isinstance(chunk, int):
            if chunk >= 2:
                _wait_s()
        else:
            pl.when(chunk >= 2)(_wait_s)

        pltpu.async_copy(row_hbm.at[pl.ds(off, n)], idx, sem_i)
        pltpu.async_copy(m_hbm.at[pl.ds(off, n)], m, sem_m)

    def drain(chunk, n, idx, m, sem_i, sem_m, sem_s):
        off = base + chunk * GC
        pltpu.make_async_copy(row_hbm.at[pl.ds(off, n)], idx, sem_i).wait()
        pltpu.make_async_copy(m_hbm.at[pl.ds(off, n)], m, sem_m).wait()
        pltpu.async_copy(m, agg_sh.at[idx], sem_s, add=True)

    fire(0, GC, idx0, m0, sem_i0, sem_m0, sem_s0)

    def step(j, carry):
        fire(2 * j + 1, GC, idx1, m1, sem_i1, sem_m1, sem_s1)
        drain(2 * j, GC, idx0, m0, sem_i0, sem_m0, sem_s0)
        fire(2 * j + 2, GC, idx0, m0, sem_i0, sem_m0, sem_s0)
        drain(2 * j + 1, GC, idx1, m1, sem_i1, sem_m1, sem_s1)
        return carry

    lax.fori_loop(0, NPAIR, step, 0)
    pltpu.async_copy(row_hbm.at[pl.ds(base + NFULL * GC, TAIL)], idxt,
                     sem_i1)
    pltpu.async_copy(m_hbm.at[pl.ds(base + NFULL * GC, TAIL)], mt, sem_m1)
    drain(NFULL - 1, GC, idx0, m0, sem_i0, sem_m0, sem_s0)
    pltpu.make_async_copy(row_hbm.at[pl.ds(base + NFULL * GC, TAIL)], idxt,
                          sem_i1).wait()
    pltpu.make_async_copy(m_hbm.at[pl.ds(base + NFULL * GC, TAIL)], mt,
                          sem_m1).wait()
    pltpu.sync_copy(mt, agg_sh.at[idxt], add=True)
    pltpu.make_async_copy(m0, agg_sh.at[idx0], sem_s0).wait()
    pltpu.make_async_copy(m1, agg_sh.at[idx1], sem_s1).wait()

    plsc.subcore_barrier()

    @pl.when(s < NS - 1)
    def _():
        pltpu.sync_copy(agg_sh.at[pl.ds(s * STRIPE, STRIPE)],
                        out_hbm.at[pl.ds(c * N + s * STRIPE, STRIPE)])

    @pl.when(s == NS - 1)
    def _():
        pltpu.sync_copy(agg_sh.at[pl.ds(s * STRIPE, STRIPE_LAST)],
                        out_hbm.at[pl.ds(c * N + s * STRIPE, STRIPE_LAST)])


def _softplus(z):
    return jnp.maximum(z, 0.0) + jnp.log1p(jnp.exp(-jnp.abs(z)))


BE = 2000
BP = 1000


def _proj_body(x_ref, w1a_ref, w1b_ref, b1_ref, p_ref, q_ref):
    xb = x_ref[...].astype(jnp.bfloat16)
    p_ref[...] = jnp.dot(xb, w1a_ref[...],
                         preferred_element_type=jnp.float32) + b1_ref[...]
    q_ref[...] = jnp.dot(xb, w1b_ref[...],
                         preferred_element_type=jnp.float32)


def _proj(xf, w1a, w1b, b1):
    grid = (N // BP,)
    blk = lambda i: (i, 0)
    full = lambda i: (0, 0)
    return pl.pallas_call(
        _proj_body,
        grid=grid,
        in_specs=[
            pl.BlockSpec((BP, D), blk),
            pl.BlockSpec((D, D), full),
            pl.BlockSpec((D, D), full),
            pl.BlockSpec((1, D), full),
        ],
        out_specs=(pl.BlockSpec((BP, D), blk), pl.BlockSpec((BP, D), blk)),
        out_shape=(jax.ShapeDtypeStruct((N, D), jnp.float32),
                   jax.ShapeDtypeStruct((N, D), jnp.float32)),
    )(xf, w1a, w1b, b1)


def _edge_mlp_body(z_ref, w2_ref, b2_ref, o_ref):
    h = _softplus(z_ref[...])
    h = jnp.dot(h.astype(jnp.bfloat16), w2_ref[...],
                preferred_element_type=jnp.float32)
    o_ref[...] = _softplus(h + b2_ref[...])


def _edge_mlp(z, w2, b2):
    grid = (EC // BE,)
    blk = lambda i: (i, 0)
    full = lambda i: (0, 0)
    return pl.pallas_call(
        _edge_mlp_body,
        grid=grid,
        in_specs=[
            pl.BlockSpec((BE, D), blk),
            pl.BlockSpec((D, D), full),
            pl.BlockSpec((1, D), full),
        ],
        out_specs=pl.BlockSpec((BE, D), blk),
        out_shape=jax.ShapeDtypeStruct((EC, D), jnp.float32),
    )(z, w2, b2)


BN = 1000


def _node_mlp_body(x_ref, g0_ref, g1_ref, w1x_ref, w1g_ref, b1_ref,
                   w2_ref, b2_ref, w3_ref, b3_ref, o_ref):
    g = g0_ref[...] + g1_ref[...]
    z = jnp.dot(x_ref[...], w1x_ref[...], preferred_element_type=jnp.float32)
    z = z + jnp.dot(g, w1g_ref[...], preferred_element_type=jnp.float32)
    h = _softplus(z + b1_ref[...])
    h = jnp.dot(h, w2_ref[...], preferred_element_type=jnp.float32)
    h = _softplus(h + b2_ref[...])
    h = jnp.dot(h, w3_ref[...], preferred_element_type=jnp.float32)
    o_ref[...] = h + b3_ref[...]


def _node_mlp(xf, g0, g1, w1x, w1g, b1, w2, b2, w3, b3):
    grid = (N // BN,)
    blk = lambda i: (i, 0)
    full = lambda i: (0, 0)
    return pl.pallas_call(
        _node_mlp_body,
        grid=grid,
        in_specs=[
            pl.BlockSpec((BN, D), blk),
            pl.BlockSpec((BN, D), blk),
            pl.BlockSpec((BN, D), blk),
            pl.BlockSpec((D, D), full),
            pl.BlockSpec((D, D), full),
            pl.BlockSpec((1, D), full),
            pl.BlockSpec((D, D), full),
            pl.BlockSpec((1, D), full),
            pl.BlockSpec((D, D), full),
            pl.BlockSpec((1, D), full),
        ],
        out_specs=pl.BlockSpec((BN, D), blk),
        out_shape=jax.ShapeDtypeStruct((N, D), jnp.float32),
    )(xf, g0, g1, w1x, w1g, b1, w2, b2, w3, b3)


def kernel(x, edge_idx, We1, be1, We2, be2, Wn1, bn1, Wn2, bn2, Wn3, bn3):
    xf = x.reshape(N, D)
    row = edge_idx[0].astype(jnp.int32)
    col = edge_idx[1].astype(jnp.int32)

    w1a = We1[:D].astype(jnp.bfloat16)
    w1b = We1[D:].astype(jnp.bfloat16)
    w2 = We2.astype(jnp.bfloat16)

    p, q = _proj(xf, w1a, w1b, be1.reshape(1, D))

    acc = jnp.zeros((2 * N, D), jnp.float32)
    for k in range(NSPLIT):
        sl = slice(k * EC, (k + 1) * EC)
        z = _sc_gather_add(p, q, row[sl], col[sl])
        m = _edge_mlp(z, w2, be2.reshape(1, D))
        acc = _sc_scatter(m, row[sl], acc)

    out = _node_mlp(xf, acc[:N], acc[N:], Wn1[:D], Wn1[D:],
                    bn1.reshape(1, D), Wn2, bn2.reshape(1, D),
                    Wn3, bn3.reshape(1, D))
    return out.reshape(1, N, D)


# speedup vs baseline: 6.1745x; 1.0254x over previous
import functools

import jax
import jax.numpy as jnp
from jax import lax
from jax.experimental import pallas as pl
from jax.experimental.pallas import tpu as pltpu
from jax.experimental.pallas import tpu_sc as plsc

N = 10000
E = 320000
D = 128

NC = 2
NS = 16
NW = NC * NS
NSPLIT = 2
EC = E // NSPLIT
EPW = EC // NW
GC = 128
NFULL = EPW // GC
TAIL = EPW - NFULL * GC
NPAIR = (NFULL - 1) // 2
STRIPE = 632
STRIPE_LAST = N - (NS - 1) * STRIPE

assert NFULL % 2 == 1 and TAIL % 4 == 0 and TAIL > 0

_sc_mesh = plsc.VectorSubcoreMesh(core_axis_name="c", subcore_axis_name="s")


@functools.partial(
    pl.kernel,
    out_type=jax.ShapeDtypeStruct((EC, D), jnp.float32),
    mesh=_sc_mesh,
    scratch_types=[
        pltpu.VMEM((EPW,), jnp.int32),
        pltpu.VMEM((EPW,), jnp.int32),
        pltpu.VMEM((GC, D), jnp.float32),
        pltpu.VMEM((GC, D), jnp.float32),
        pltpu.VMEM((GC, D), jnp.float32),
        pltpu.VMEM((GC, D), jnp.float32),
        pltpu.VMEM((TAIL, D), jnp.float32),
        pltpu.VMEM((TAIL, D), jnp.float32),
        pltpu.VMEM((GC,), jnp.int32),
        pltpu.VMEM((GC,), jnp.int32),
        pltpu.VMEM_SHARED((NS * 2 * GC, D), jnp.float32),
        pltpu.SemaphoreType.DMA,
        pltpu.SemaphoreType.DMA,
        pltpu.SemaphoreType.DMA,
        pltpu.SemaphoreType.DMA,
        pltpu.SemaphoreType.DMA,
        pltpu.SemaphoreType.DMA,
    ],
)
def _sc_gather_add(p_hbm, q_hbm, row_hbm, col_hbm, z_hbm,
                   ridx_all, cidx_all, arows0, brows0, arows1, brows1,
                   arowst, browst, ident0, ident1, zsp,
                   sem_a0, sem_b0, sem_a1, sem_b1, sem_w0, sem_w1):
    c = lax.axis_index("c")
    s = lax.axis_index("s")
    wid = s * NC + c
    base = wid * EPW

    pltpu.sync_copy(row_hbm.at[pl.ds(base, EPW)], ridx_all)
    pltpu.sync_copy(col_hbm.at[pl.ds(base, EPW)], cidx_all)

    for k in range(GC // 16):
        off16 = lax.iota(jnp.int32, 16) + (16 * k) + s * (2 * GC)
        ident0[pl.ds(16 * k, 16)] = off16
        ident1[pl.ds(16 * k, 16)] = off16 + GC

    def fire(chunk, n, arows, brows, sem_a, sem_b):
        coff = chunk * GC
        pltpu.async_copy(p_hbm.at[ridx_all.at[pl.ds(coff, n)]], arows, sem_a)
        pltpu.async_copy(q_hbm.at[cidx_all.at[pl.ds(coff, n)]], brows, sem_b)

    def drain(chunk, par, arows, brows, ident, sem_a, sem_b, sem_w):
        off = base + chunk * GC
        coff = chunk * GC
        zoff = s * (2 * GC) + par * GC
        pltpu.make_async_copy(p_hbm.at[ridx_all.at[pl.ds(coff, GC)]],
                              arows, sem_a).wait()
        pltpu.make_async_copy(q_hbm.at[cidx_all.at[pl.ds(coff, GC)]],
                              brows, sem_b).wait()

        def _wait_wb():
            pltpu.make_async_copy(zsp.at[pl.ds(zoff, GC)],
                                  z_hbm.at[pl.ds(off, GC)], sem_w).wait()

        if isinstance(chunk, int):
            if chunk >= 2:
                _wait_wb()
        else:
            pl.when(chunk >= 2)(_wait_wb)

        pltpu.sync_copy(arows, zsp.at[pl.ds(zoff, GC)])
        pltpu.sync_copy(brows, zsp.at[ident], add=True)
        pltpu.async_copy(zsp.at[pl.ds(zoff, GC)], z_hbm.at[pl.ds(off, GC)],
                         sem_w)

    fire(0, GC, arows0, brows0, sem_a0, sem_b0)

    def step(j, carry):
        fire(2 * j + 1, GC, arows1, brows1, sem_a1, sem_b1)
        drain(2 * j, 0, arows0, brows0, ident0, sem_a0, sem_b0, sem_w0)
        fire(2 * j + 2, GC, arows0, brows0, sem_a0, sem_b0)
        drain(2 * j + 1, 1, arows1, brows1, ident1, sem_a1, sem_b1, sem_w1)
        return carry

    lax.fori_loop(0, NPAIR, step, 0)
    fire(NFULL, TAIL, arowst, browst, sem_a1, sem_b1)
    drain(NFULL - 1, 0, arows0, brows0, ident0, sem_a0, sem_b0, sem_w0)
    off_t = base + NFULL * GC
    coff_t = NFULL * GC
    pltpu.make_async_copy(p_hbm.at[ridx_all.at[pl.ds(coff_t, TAIL)]],
                          arowst, sem_a1).wait()
    pltpu.make_async_copy(q_hbm.at[cidx_all.at[pl.ds(coff_t, TAIL)]],
                          browst, sem_b1).wait()
    for r in range(TAIL):
        for k in range(D // 16):
            sl = pl.ds(16 * k, 16)
            arowst[r, sl] = arowst[r, sl] + browst[r, sl]
    pltpu.sync_copy(arowst, z_hbm.at[pl.ds(off_t, TAIL)])
    pltpu.make_async_copy(zsp.at[pl.ds(s * (2 * GC), GC)],
                          z_hbm.at[pl.ds(base, GC)], sem_w0).wait()
    pltpu.make_async_copy(zsp.at[pl.ds(s * (2 * GC) + GC, GC)],
                          z_hbm.at[pl.ds(base, GC)], sem_w1).wait()


@functools.partial(
    pl.kernel,
    out_type=jax.ShapeDtypeStruct((2 * N, D), jnp.float32),
    mesh=_sc_mesh,
    scratch_types=[
        pltpu.VMEM((GC,), jnp.int32),
        pltpu.VMEM((GC, D), jnp.float32),
        pltpu.VMEM((GC,), jnp.int32),
        pltpu.VMEM((GC, D), jnp.float32),
        pltpu.VMEM((TAIL,), jnp.int32),
        pltpu.VMEM((TAIL, D), jnp.float32),
        pltpu.VMEM_SHARED((N, D), jnp.float32),
        pltpu.SemaphoreType.DMA,
        pltpu.SemaphoreType.DMA,
        pltpu.SemaphoreType.DMA,
        pltpu.SemaphoreType.DMA,
        pltpu.SemaphoreType.DMA,
        pltpu.SemaphoreType.DMA,
    ],
)
def _sc_scatter(m_hbm, row_hbm, init_hbm, out_hbm,
                idx0, m0, idx1, m1, idxt, mt, agg_sh,
                sem_i0, sem_m0, sem_i1, sem_m1, sem_s0, sem_s1):
    c = lax.axis_index("c")
    s = lax.axis_index("s")
    wid = s * NC + c
    base = wid * EPW

    @pl.when(s < NS - 1)
    def _():
        pltpu.sync_copy(init_hbm.at[pl.ds(c * N + s * STRIPE, STRIPE)],
                        agg_sh.at[pl.ds(s * STRIPE, STRIPE)])

    @pl.when(s == NS - 1)
    def _():
        pltpu.sync_copy(init_hbm.at[pl.ds(c * N + s * STRIPE, STRIPE_LAST)],
                        agg_sh.at[pl.ds(s * STRIPE, STRIPE_LAST)])

    plsc.subcore_barrier()

    def fire(chunk, n, idx, m, sem_i, sem_m, sem_s):
        off = base + chunk * GC

        def _wait_s():
            pltpu.make_async_copy(m, agg_sh.at[idx], sem_s).wait()

        if isinstance(chunk, int):
            if chunk >= 2:
                _wait_s()
        else:
            pl.when(chunk >= 2)(_wait_s)

        pltpu.async_copy(row_hbm.at[pl.ds(off, n)], idx, sem_i)
        pltpu.async_copy(m_hbm.at[pl.ds(off, n)], m, sem_m)

    def drain(chunk, n, idx, m, sem_i, sem_m, sem_s):
        off = base + chunk * GC
        pltpu.make_async_copy(row_hbm.at[pl.ds(off, n)], idx, sem_i).wait()
        pltpu.make_async_copy(m_hbm.at[pl.ds(off, n)], m, sem_m).wait()
        pltpu.async_copy(m, agg_sh.at[idx], sem_s, add=True)

    fire(0, GC, idx0, m0, sem_i0, sem_m0, sem_s0)

    def step(j, carry):
        fire(2 * j + 1, GC, idx1, m1, sem_i1, sem_m1, sem_s1)
        drain(2 * j, GC, idx0, m0, sem_i0, sem_m0, sem_s0)
        fire(2 * j + 2, GC, idx0, m0, sem_i0, sem_m0, sem_s0)
        drain(2 * j + 1, GC, idx1, m1, sem_i1, sem_m1, sem_s1)
        return carry

    lax.fori_loop(0, NPAIR, step, 0)
    pltpu.async_copy(row_hbm.at[pl.ds(base + NFULL * GC, TAIL)], idxt,
                     sem_i1)
    pltpu.async_copy(m_hbm.at[pl.ds(base + NFULL * GC, TAIL)], mt, sem_m1)
    drain(NFULL - 1, GC, idx0, m0, sem_i0, sem_m0, sem_s0)
    pltpu.make_async_copy(row_hbm.at[pl.ds(base + NFULL * GC, TAIL)], idxt,
                          sem_i1).wait()
    pltpu.make_async_copy(m_hbm.at[pl.ds(base + NFULL * GC, TAIL)], mt,
                          sem_m1).wait()
    pltpu.sync_copy(mt, agg_sh.at[idxt], add=True)
    pltpu.make_async_copy(m0, agg_sh.at[idx0], sem_s0).wait()
    pltpu.make_async_copy(m1, agg_sh.at[idx1], sem_s1).wait()

    plsc.subcore_barrier()

    @pl.when(s < NS - 1)
    def _():
        pltpu.sync_copy(agg_sh.at[pl.ds(s * STRIPE, STRIPE)],
                        out_hbm.at[pl.ds(c * N + s * STRIPE, STRIPE)])

    @pl.when(s == NS - 1)
    def _():
        pltpu.sync_copy(agg_sh.at[pl.ds(s * STRIPE, STRIPE_LAST)],
                        out_hbm.at[pl.ds(c * N + s * STRIPE, STRIPE_LAST)])


def _softplus(z):
    return jnp.maximum(z, 0.0) + jnp.log1p(jnp.exp(-jnp.abs(z)))


BE = 2000
BP = 1000


def _proj_body(x_ref, w1a_ref, w1b_ref, b1_ref, p_ref, q_ref):
    xb = x_ref[...].astype(jnp.bfloat16)
    p_ref[...] = jnp.dot(xb, w1a_ref[...],
                         preferred_element_type=jnp.float32) + b1_ref[...]
    q_ref[...] = jnp.dot(xb, w1b_ref[...],
                         preferred_element_type=jnp.float32)


def _proj(xf, w1a, w1b, b1):
    grid = (N // BP,)
    blk = lambda i: (i, 0)
    full = lambda i: (0, 0)
    return pl.pallas_call(
        _proj_body,
        grid=grid,
        in_specs=[
            pl.BlockSpec((BP, D), blk),
            pl.BlockSpec((D, D), full),
            pl.BlockSpec((D, D), full),
            pl.BlockSpec((1, D), full),
        ],
        out_specs=(pl.BlockSpec((BP, D), blk), pl.BlockSpec((BP, D), blk)),
        out_shape=(jax.ShapeDtypeStruct((N, D), jnp.float32),
                   jax.ShapeDtypeStruct((N, D), jnp.float32)),
    )(xf, w1a, w1b, b1)


def _edge_mlp_body(z_ref, w2_ref, b2_ref, o_ref):
    h = _softplus(z_ref[...])
    h = jnp.dot(h.astype(jnp.bfloat16), w2_ref[...],
                preferred_element_type=jnp.float32)
    o_ref[...] = _softplus(h + b2_ref[...])


def _edge_mlp(z, w2, b2):
    grid = (EC // BE,)
    blk = lambda i: (i, 0)
    full = lambda i: (0, 0)
    return pl.pallas_call(
        _edge_mlp_body,
        grid=grid,
        in_specs=[
            pl.BlockSpec((BE, D), blk),
            pl.BlockSpec((D, D), full),
            pl.BlockSpec((1, D), full),
        ],
        out_specs=pl.BlockSpec((BE, D), blk),
        out_shape=jax.ShapeDtypeStruct((EC, D), jnp.float32),
    )(z, w2, b2)


BN = 1000


def _node_mlp_body(x_ref, g0_ref, g1_ref, w1x_ref, w1g_ref, b1_ref,
                   w2_ref, b2_ref, w3_ref, b3_ref, o_ref):
    g = g0_ref[...] + g1_ref[...]
    z = jnp.dot(x_ref[...], w1x_ref[...], preferred_element_type=jnp.float32)
    z = z + jnp.dot(g, w1g_ref[...], preferred_element_type=jnp.float32)
    h = _softplus(z + b1_ref[...])
    h = jnp.dot(h, w2_ref[...], preferred_element_type=jnp.float32)
    h = _softplus(h + b2_ref[...])
    h = jnp.dot(h, w3_ref[...], preferred_element_type=jnp.float32)
    o_ref[...] = h + b3_ref[...]


def _node_mlp(xf, g0, g1, w1x, w1g, b1, w2, b2, w3, b3):
    grid = (N // BN,)
    blk = lambda i: (i, 0)
    full = lambda i: (0, 0)
    return pl.pallas_call(
        _node_mlp_body,
        grid=grid,
        in_specs=[
            pl.BlockSpec((BN, D), blk),
            pl.BlockSpec((BN, D), blk),
            pl.BlockSpec((BN, D), blk),
            pl.BlockSpec((D, D), full),
            pl.BlockSpec((D, D), full),
            pl.BlockSpec((1, D), full),
            pl.BlockSpec((D, D), full),
            pl.BlockSpec((1, D), full),
            pl.BlockSpec((D, D), full),
            pl.BlockSpec((1, D), full),
        ],
        out_specs=pl.BlockSpec((BN, D), blk),
        out_shape=jax.ShapeDtypeStruct((N, D), jnp.float32),
    )(xf, g0, g1, w1x, w1g, b1, w2, b2, w3, b3)


def kernel(x, edge_idx, We1, be1, We2, be2, Wn1, bn1, Wn2, bn2, Wn3, bn3):
    xf = x.reshape(N, D)
    row = edge_idx[0].astype(jnp.int32)
    col = edge_idx[1].astype(jnp.int32)

    w1a = We1[:D].astype(jnp.bfloat16)
    w1b = We1[D:].astype(jnp.bfloat16)
    w2 = We2.astype(jnp.bfloat16)

    p, q = _proj(xf, w1a, w1b, be1.reshape(1, D))

    acc = jnp.zeros((2 * N, D), jnp.float32)
    for k in range(NSPLIT):
        sl = slice(k * EC, (k + 1) * EC)
        z = _sc_gather_add(p, q, row[sl], col[sl])
        m = _edge_mlp(z, w2, be2.reshape(1, D))
        acc = _sc_scatter(m, row[sl], acc)

    out = _node_mlp(xf, acc[:N], acc[N:], Wn1[:D], Wn1[D:],
                    bn1.reshape(1, D), Wn2, bn2.reshape(1, D),
                    Wn3, bn3.reshape(1, D))
    return out.reshape(1, N, D)
